# baseline (device time: 30784 ns/iter reference)
import jax
import jax.numpy as jnp
from jax import lax
from jax.experimental import pallas as pl
from jax.experimental.pallas import tpu as pltpu

N_DEV = 4
N_TOK = 512
D_IN = 256
D_OUT = 512
N_EXP = 8
E_PER = N_EXP // N_DEV


def kernel(x, router_W, route_idx, expert_W):
    def body(x_ref, rw_ref, idx_ref, ew_ref, out_ref,
             comm_ref, send_sems, recv_sems):
        my = lax.axis_index("i")
        left = lax.rem(my + N_DEV - 1, N_DEV)
        right = lax.rem(my + 1, N_DEV)

        barrier_sem = pltpu.get_barrier_semaphore()
        for nbr in (left, right):
            pl.semaphore_signal(
                barrier_sem, inc=1,
                device_id=(nbr,), device_id_type=pl.DeviceIdType.MESH,
            )
        pl.semaphore_wait(barrier_sem, 2)

        xv = x_ref[:, :]
        scores = jnp.dot(xv, rw_ref[:, :],
                         preferred_element_type=jnp.float32)
        m = jnp.max(scores, axis=-1, keepdims=True)
        p = jnp.exp(scores - m)
        p = p / jnp.sum(p, axis=-1, keepdims=True)

        idx = idx_ref[:, :]
        idx0 = idx[:, 0:1]
        idx1 = idx[:, 1:2]
        eids = lax.broadcasted_iota(jnp.int32, (N_TOK, N_EXP), 1)
        p0 = jnp.sum(jnp.where(eids == idx0, p, 0.0), axis=-1,
                     keepdims=True)
        p1 = jnp.sum(jnp.where(eids == idx1, p, 0.0), axis=-1,
                     keepdims=True)
        gs = p0 + p1

        xb = xv.astype(jnp.bfloat16)
        acc = jnp.zeros((N_TOK, D_OUT), jnp.float32)
        for j in range(E_PER):
            e = my * E_PER + j
            ge = (jnp.where(idx0 == e, p0, 0.0)
                  + jnp.where(idx1 == e, p1, 0.0)) / gs
            y = jnp.dot(xb, ew_ref[j].astype(jnp.bfloat16),
                        preferred_element_type=jnp.float32)
            acc = acc + ge * y
        comm_ref[my] = acc.astype(jnp.bfloat16)

        for h in range(N_DEV - 1):
            send_origin = lax.rem(my + N_DEV - h, N_DEV)
            recv_origin = lax.rem(my + N_DEV - h - 1, N_DEV)
            send = pltpu.make_async_remote_copy(
                src_ref=comm_ref.at[send_origin],
                dst_ref=comm_ref.at[send_origin],
                send_sem=send_sems.at[h],
                recv_sem=recv_sems.at[send_origin],
                device_id=(right,),
                device_id_type=pl.DeviceIdType.MESH,
            )
            send.start()
            recv = pltpu.make_async_remote_copy(
                src_ref=comm_ref.at[recv_origin],
                dst_ref=comm_ref.at[recv_origin],
                send_sem=send_sems.at[h],
                recv_sem=recv_sems.at[recv_origin],
                device_id=(right,),
                device_id_type=pl.DeviceIdType.MESH,
            )
            recv.wait_recv()
            send.wait_send()

        out_ref[:, :] = (comm_ref[0].astype(jnp.float32)
                         + comm_ref[1].astype(jnp.float32)
                         + comm_ref[2].astype(jnp.float32)
                         + comm_ref[3].astype(jnp.float32))

    return pl.pallas_call(
        body,
        out_shape=jax.ShapeDtypeStruct((N_TOK, D_OUT), jnp.float32),
        in_specs=[
            pl.BlockSpec(memory_space=pltpu.VMEM),
            pl.BlockSpec(memory_space=pltpu.VMEM),
            pl.BlockSpec(memory_space=pltpu.VMEM),
            pl.BlockSpec(memory_space=pltpu.VMEM),
        ],
        out_specs=pl.BlockSpec(memory_space=pltpu.VMEM),
        scratch_shapes=[
            pltpu.VMEM((N_DEV, N_TOK, D_OUT), jnp.bfloat16),
            pltpu.SemaphoreType.DMA((N_DEV - 1,)),
            pltpu.SemaphoreType.DMA((N_DEV,)),
        ],
        compiler_params=pltpu.CompilerParams(collective_id=0),
    )(x, router_W, route_idx, expert_W)


# device time: 21907 ns/iter; 1.4052x vs baseline; 1.4052x over previous
import jax
import jax.numpy as jnp
from jax import lax
from jax.experimental import pallas as pl
from jax.experimental.pallas import tpu as pltpu

N_DEV = 4
N_TOK = 512
D_IN = 256
D_OUT = 512
N_EXP = 8
E_PER = N_EXP // N_DEV


def kernel(x, router_W, route_idx, expert_W):
    def body(x_ref, rw_ref, idx_ref, ew_ref, out_ref,
             comm_ref, send_sems, recv_sems):
        my = lax.axis_index("i")
        left = lax.rem(my + N_DEV - 1, N_DEV)
        right = lax.rem(my + 1, N_DEV)

        barrier_sem = pltpu.get_barrier_semaphore()
        for k in range(1, N_DEV):
            pl.semaphore_signal(
                barrier_sem, inc=1,
                device_id=(lax.rem(my + k, N_DEV),),
                device_id_type=pl.DeviceIdType.MESH,
            )
        pl.semaphore_wait(barrier_sem, N_DEV - 1)

        xv = x_ref[:, :]
        scores = jnp.dot(xv, rw_ref[:, :],
                         preferred_element_type=jnp.float32)
        m = jnp.max(scores, axis=-1, keepdims=True)
        p = jnp.exp(scores - m)
        p = p / jnp.sum(p, axis=-1, keepdims=True)

        idx = idx_ref[:, :]
        idx0 = idx[:, 0:1]
        idx1 = idx[:, 1:2]
        eids = lax.broadcasted_iota(jnp.int32, (N_TOK, N_EXP), 1)
        p0 = jnp.sum(jnp.where(eids == idx0, p, 0.0), axis=-1,
                     keepdims=True)
        p1 = jnp.sum(jnp.where(eids == idx1, p, 0.0), axis=-1,
                     keepdims=True)
        gs = p0 + p1

        xb = xv.astype(jnp.bfloat16)
        acc = jnp.zeros((N_TOK, D_OUT), jnp.float32)
        for j in range(E_PER):
            e = my * E_PER + j
            ge = (jnp.where(idx0 == e, p0, 0.0)
                  + jnp.where(idx1 == e, p1, 0.0)) / gs
            y = jnp.dot(xb, ew_ref[j].astype(jnp.bfloat16),
                        preferred_element_type=jnp.float32)
            acc = acc + ge * y
        comm_ref[my] = acc.astype(jnp.bfloat16)

        sends = []
        for k in range(1, N_DEV):
            dst = lax.rem(my + k, N_DEV)
            send = pltpu.make_async_remote_copy(
                src_ref=comm_ref.at[my],
                dst_ref=comm_ref.at[my],
                send_sem=send_sems.at[k - 1],
                recv_sem=recv_sems.at[my],
                device_id=(dst,),
                device_id_type=pl.DeviceIdType.MESH,
            )
            send.start()
            sends.append(send)
        for k in range(1, N_DEV):
            origin = lax.rem(my + k, N_DEV)
            recv = pltpu.make_async_remote_copy(
                src_ref=comm_ref.at[origin],
                dst_ref=comm_ref.at[origin],
                send_sem=send_sems.at[k - 1],
                recv_sem=recv_sems.at[origin],
                device_id=(right,),
                device_id_type=pl.DeviceIdType.MESH,
            )
            recv.wait_recv()
        for send in sends:
            send.wait_send()

        out_ref[:, :] = (comm_ref[0].astype(jnp.float32)
                         + comm_ref[1].astype(jnp.float32)
                         + comm_ref[2].astype(jnp.float32)
                         + comm_ref[3].astype(jnp.float32))

    return pl.pallas_call(
        body,
        out_shape=jax.ShapeDtypeStruct((N_TOK, D_OUT), jnp.float32),
        in_specs=[
            pl.BlockSpec(memory_space=pltpu.VMEM),
            pl.BlockSpec(memory_space=pltpu.VMEM),
            pl.BlockSpec(memory_space=pltpu.VMEM),
            pl.BlockSpec(memory_space=pltpu.VMEM),
        ],
        out_specs=pl.BlockSpec(memory_space=pltpu.VMEM),
        scratch_shapes=[
            pltpu.VMEM((N_DEV, N_TOK, D_OUT), jnp.bfloat16),
            pltpu.SemaphoreType.DMA((N_DEV - 1,)),
            pltpu.SemaphoreType.DMA((N_DEV,)),
        ],
        compiler_params=pltpu.CompilerParams(collective_id=0),
    )(x, router_W, route_idx, expert_W)
